# baseline (speedup 1.0000x reference)
import jax
import jax.numpy as jnp
from jax.experimental import pallas as pl
from jax.experimental.pallas import tpu as pltpu

NFEAT = 256
NHID = 32
NCLASS = 64
NHEADS = 8
ALPHA = 0.2
B = 4
N = 512

_NT = (((1,), (1,)), ((), ()))


def _elu(v):
    return jnp.where(v > 0, v, jnp.exp(jnp.minimum(v, 0.0)) - 1.0)


def _att_prop(p, pa, q, qa, h, mask, d):
    e = jnp.minimum(p * q, pa * qa) * mask
    lane = jax.lax.broadcasted_iota(jnp.int32, (N, d), 1)
    ones_col = (lane == 0).astype(jnp.float32)
    rhs = jnp.concatenate([h, ones_col], axis=1)
    acc = jnp.dot(e, rhs.astype(jnp.bfloat16),
                  preferred_element_type=jnp.float32)
    hp = acc[:, :d]
    rowsum = acc[:, d:d + 1]
    return hp * (1.0 / rowsum)


def _gat_body(x_ref, adj_ref, watt_ref, aatt_ref, wout_ref, aout_ref, out_ref,
              xbuf, sem):
    pltpu.make_async_copy(x_ref.at[0], xbuf.at[0], sem.at[0]).start()

    mask = (adj_ref[...] != 0).astype(jnp.bfloat16)

    wcat = jnp.concatenate([watt_ref[hi] for hi in range(NHEADS)],
                           axis=1)

    aattT = aatt_ref[:, 0, :].T
    row_head = jax.lax.broadcasted_iota(
        jnp.int32, (NHEADS * NHID, NHEADS), 0) // NHID
    col_head = jax.lax.broadcasted_iota(
        jnp.int32, (NHEADS * NHID, NHEADS), 1)
    blk = (row_head == col_head).astype(jnp.float32)
    acomb = jnp.concatenate(
        [jnp.tile(aattT[:NHID, :], (NHEADS, 1)) * blk,
         jnp.tile(aattT[NHID:, :], (NHEADS, 1)) * blk],
        axis=1)

    a2_src = aout_ref[:, :NCLASS]
    a2_dst = aout_ref[:, NCLASS:]

    for b in range(B):
        if b + 1 < B:
            pltpu.make_async_copy(
                x_ref.at[b + 1], xbuf.at[(b + 1) % 2],
                sem.at[(b + 1) % 2]).start()
        pltpu.make_async_copy(x_ref.at[b], xbuf.at[b % 2], sem.at[b % 2]).wait()
        xb = xbuf[b % 2]
        h_all = jnp.dot(xb, wcat, preferred_element_type=jnp.float32)
        fg = jnp.dot(h_all, acomb,
                     preferred_element_type=jnp.float32)
        fgT = fg.T
        P = jnp.exp(-fg[:, :NHEADS]).astype(jnp.bfloat16)
        PA = jnp.exp(-ALPHA * fg[:, :NHEADS]).astype(jnp.bfloat16)
        Q = jnp.exp(-fgT[NHEADS:, :]).astype(jnp.bfloat16)
        QA = jnp.exp(-ALPHA * fgT[NHEADS:, :]).astype(jnp.bfloat16)

        head_outs = []
        for hi in range(NHEADS):
            h = h_all[:, hi * NHID:(hi + 1) * NHID]
            head_outs.append(_elu(_att_prop(
                P[:, hi:hi + 1], PA[:, hi:hi + 1],
                Q[hi:hi + 1, :], QA[hi:hi + 1, :], h, mask, NHID)))
        x1 = jnp.concatenate(head_outs, axis=1)

        h2 = jnp.dot(x1, wout_ref[...], preferred_element_type=jnp.float32)
        f2 = jax.lax.dot_general(h2, a2_src, _NT,
                                 preferred_element_type=jnp.float32)
        g2T = jax.lax.dot_general(a2_dst, h2, _NT,
                                  preferred_element_type=jnp.float32)
        out_ref[b] = _elu(_att_prop(
            jnp.exp(-f2).astype(jnp.bfloat16),
            jnp.exp(-ALPHA * f2).astype(jnp.bfloat16),
            jnp.exp(-g2T).astype(jnp.bfloat16),
            jnp.exp(-ALPHA * g2T).astype(jnp.bfloat16), h2, mask, NCLASS))


def kernel(x, adj, W_att, a_att, W_out, a_out):
    return pl.pallas_call(
        _gat_body,
        in_specs=[
            pl.BlockSpec(memory_space=pl.ANY),
            pl.BlockSpec(memory_space=pltpu.VMEM),
            pl.BlockSpec(memory_space=pltpu.VMEM),
            pl.BlockSpec(memory_space=pltpu.VMEM),
            pl.BlockSpec(memory_space=pltpu.VMEM),
            pl.BlockSpec(memory_space=pltpu.VMEM),
        ],
        out_specs=pl.BlockSpec(memory_space=pltpu.VMEM),
        scratch_shapes=[
            pltpu.VMEM((2, N, NFEAT), jnp.float32),
            pltpu.SemaphoreType.DMA((2,)),
        ],
        out_shape=jax.ShapeDtypeStruct((B, N, NCLASS), jnp.float32),
    )(x, adj, W_att, a_att, W_out, a_out)


# speedup vs baseline: 1.1201x; 1.1201x over previous
import jax
import jax.numpy as jnp
from jax.experimental import pallas as pl

NFEAT = 256
NHID = 32
NCLASS = 64
NHEADS = 8
ALPHA = 0.2
B = 4
N = 512

_NT = (((1,), (1,)), ((), ()))


def _elu(v):
    return jnp.where(v > 0, v, jnp.exp(jnp.minimum(v, 0.0)) - 1.0)


def _att_prop(p, pa, q, qa, h, mask, d):
    e = jnp.minimum(p * q, pa * qa) * mask
    lane = jax.lax.broadcasted_iota(jnp.int32, (N, d), 1)
    ones_col = (lane == 0).astype(jnp.float32)
    rhs = jnp.concatenate([h, ones_col], axis=1)
    acc = jnp.dot(e, rhs.astype(jnp.bfloat16),
                  preferred_element_type=jnp.float32)
    hp = acc[:, :d]
    rowsum = acc[:, d:d + 1]
    return hp * (1.0 / rowsum)


def _gat_body(x_ref, adj_ref, watt_ref, aatt_ref, wout_ref, aout_ref, out_ref):
    mask = (adj_ref[...] != 0).astype(jnp.bfloat16)

    wcat = jnp.concatenate([watt_ref[hi] for hi in range(NHEADS)],
                           axis=1)

    aattT = aatt_ref[:, 0, :].T
    row_head = jax.lax.broadcasted_iota(
        jnp.int32, (NHEADS * NHID, NHEADS), 0) // NHID
    col_head = jax.lax.broadcasted_iota(
        jnp.int32, (NHEADS * NHID, NHEADS), 1)
    blk = (row_head == col_head).astype(jnp.float32)
    acomb = jnp.concatenate(
        [jnp.tile(aattT[:NHID, :], (NHEADS, 1)) * blk,
         jnp.tile(aattT[NHID:, :], (NHEADS, 1)) * blk],
        axis=1)

    a2_src = aout_ref[:, :NCLASS]
    a2_dst = aout_ref[:, NCLASS:]

    for b in range(B):
        xb = x_ref[b]
        h_all = jnp.dot(xb, wcat, preferred_element_type=jnp.float32)
        fg = jnp.dot(h_all, acomb,
                     preferred_element_type=jnp.float32)
        fgT = fg.T
        P = jnp.exp(-fg[:, :NHEADS]).astype(jnp.bfloat16)
        PA = jnp.exp(-ALPHA * fg[:, :NHEADS]).astype(jnp.bfloat16)
        Q = jnp.exp(-fgT[NHEADS:, :]).astype(jnp.bfloat16)
        QA = jnp.exp(-ALPHA * fgT[NHEADS:, :]).astype(jnp.bfloat16)

        head_outs = []
        for hi in range(NHEADS):
            h = h_all[:, hi * NHID:(hi + 1) * NHID]
            head_outs.append(_elu(_att_prop(
                P[:, hi:hi + 1], PA[:, hi:hi + 1],
                Q[hi:hi + 1, :], QA[hi:hi + 1, :], h, mask, NHID)))
        x1 = jnp.concatenate(head_outs, axis=1)

        h2 = jnp.dot(x1, wout_ref[...], preferred_element_type=jnp.float32)
        f2 = jax.lax.dot_general(h2, a2_src, _NT,
                                 preferred_element_type=jnp.float32)
        g2T = jax.lax.dot_general(a2_dst, h2, _NT,
                                  preferred_element_type=jnp.float32)
        out_ref[b] = _elu(_att_prop(
            jnp.exp(-f2).astype(jnp.bfloat16),
            jnp.exp(-ALPHA * f2).astype(jnp.bfloat16),
            jnp.exp(-g2T).astype(jnp.bfloat16),
            jnp.exp(-ALPHA * g2T).astype(jnp.bfloat16), h2, mask, NCLASS))


def kernel(x, adj, W_att, a_att, W_out, a_out):
    return pl.pallas_call(
        _gat_body,
        out_shape=jax.ShapeDtypeStruct((B, N, NCLASS), jnp.float32),
    )(x, adj, W_att, a_att, W_out, a_out)


# speedup vs baseline: 1.1220x; 1.0017x over previous
import jax
import jax.numpy as jnp
from jax.experimental import pallas as pl

NFEAT = 256
NHID = 32
NCLASS = 64
NHEADS = 8
ALPHA = 0.2
B = 4
N = 512

_NT = (((1,), (1,)), ((), ()))


def _elu(v):
    return jnp.where(v > 0, v, jnp.exp(v) - 1.0)


def _att_prop(p, pa, q, qa, h, mask, d):
    e = jnp.minimum(p * q, pa * qa) * mask
    lane = jax.lax.broadcasted_iota(jnp.int32, (N, d), 1)
    ones_col = (lane == 0).astype(jnp.float32)
    rhs = jnp.concatenate([h, ones_col], axis=1)
    acc = jnp.dot(e, rhs.astype(jnp.bfloat16),
                  preferred_element_type=jnp.float32)
    hp = acc[:, :d]
    rowsum = acc[:, d:d + 1]
    return hp * (1.0 / rowsum)


def _gat_body(x_ref, adj_ref, watt_ref, aatt_ref, wout_ref, aout_ref, out_ref):
    mask = (adj_ref[...] != 0).astype(jnp.bfloat16)

    wcat = jnp.concatenate([watt_ref[hi] for hi in range(NHEADS)],
                           axis=1)

    aattT = aatt_ref[:, 0, :].T
    row_head = jax.lax.broadcasted_iota(
        jnp.int32, (NHEADS * NHID, NHEADS), 0) // NHID
    col_head = jax.lax.broadcasted_iota(
        jnp.int32, (NHEADS * NHID, NHEADS), 1)
    blk = (row_head == col_head).astype(jnp.float32)
    acomb = jnp.concatenate(
        [jnp.tile(aattT[:NHID, :], (NHEADS, 1)) * blk,
         jnp.tile(aattT[NHID:, :], (NHEADS, 1)) * blk],
        axis=1)

    a2_src = aout_ref[:, :NCLASS]
    a2_dst = aout_ref[:, NCLASS:]

    for b in range(B):
        xb = x_ref[b]
        h_all = jnp.dot(xb, wcat, preferred_element_type=jnp.float32)
        fg = jnp.dot(h_all, acomb,
                     preferred_element_type=jnp.float32)
        fgT = fg.T
        P = jnp.exp(-fg[:, :NHEADS]).astype(jnp.bfloat16)
        PA = jnp.exp(-ALPHA * fg[:, :NHEADS]).astype(jnp.bfloat16)
        Q = jnp.exp(-fgT[NHEADS:, :]).astype(jnp.bfloat16)
        QA = jnp.exp(-ALPHA * fgT[NHEADS:, :]).astype(jnp.bfloat16)

        head_outs = []
        for hi in range(NHEADS):
            h = h_all[:, hi * NHID:(hi + 1) * NHID]
            head_outs.append(_elu(_att_prop(
                P[:, hi:hi + 1], PA[:, hi:hi + 1],
                Q[hi:hi + 1, :], QA[hi:hi + 1, :], h, mask, NHID)))
        x1 = jnp.concatenate(head_outs, axis=1)

        h2 = jnp.dot(x1, wout_ref[...], preferred_element_type=jnp.float32)
        f2 = jax.lax.dot_general(h2, a2_src, _NT,
                                 preferred_element_type=jnp.float32)
        g2T = jax.lax.dot_general(a2_dst, h2, _NT,
                                  preferred_element_type=jnp.float32)
        out_ref[b] = _elu(_att_prop(
            jnp.exp(-f2).astype(jnp.bfloat16),
            jnp.exp(-ALPHA * f2).astype(jnp.bfloat16),
            jnp.exp(-g2T).astype(jnp.bfloat16),
            jnp.exp(-ALPHA * g2T).astype(jnp.bfloat16), h2, mask, NCLASS))


def kernel(x, adj, W_att, a_att, W_out, a_out):
    return pl.pallas_call(
        _gat_body,
        out_shape=jax.ShapeDtypeStruct((B, N, NCLASS), jnp.float32),
    )(x, adj, W_att, a_att, W_out, a_out)


# speedup vs baseline: 1.1231x; 1.0009x over previous
import jax
import jax.numpy as jnp
from jax.experimental import pallas as pl

NFEAT = 256
NHID = 32
NCLASS = 64
NHEADS = 8
ALPHA = 0.2
B = 4
N = 512

_NT = (((1,), (1,)), ((), ()))


def _elu(v):
    return jnp.where(v > 0, v, jnp.exp(v) - 1.0)


def _ones_col(d):
    lane = jax.lax.broadcasted_iota(jnp.int32, (N, d), 1)
    return (lane == 0).astype(jnp.float32)


def _att_prop(p, pa, q, qa, h, mask, ones_col, d):
    e = jnp.minimum(p * q, pa * qa) * mask
    rhs = jnp.concatenate([h, ones_col], axis=1)
    acc = jnp.dot(e, rhs.astype(jnp.bfloat16),
                  preferred_element_type=jnp.float32)
    hp = acc[:, :d]
    rowsum = acc[:, d:d + 1]
    return hp * (1.0 / rowsum)


def _gat_body(x_ref, adj_ref, watt_ref, aatt_ref, wout_ref, aout_ref, out_ref):
    mask = (adj_ref[...] != 0).astype(jnp.bfloat16)

    wcat = jnp.concatenate([watt_ref[hi] for hi in range(NHEADS)],
                           axis=1)

    aattT = aatt_ref[:, 0, :].T
    row_head = jax.lax.broadcasted_iota(
        jnp.int32, (NHEADS * NHID, NHEADS), 0) // NHID
    col_head = jax.lax.broadcasted_iota(
        jnp.int32, (NHEADS * NHID, NHEADS), 1)
    blk = (row_head == col_head).astype(jnp.float32)
    acomb = jnp.concatenate(
        [jnp.tile(aattT[:NHID, :], (NHEADS, 1)) * blk,
         jnp.tile(aattT[NHID:, :], (NHEADS, 1)) * blk],
        axis=1)

    a2_src = aout_ref[:, :NCLASS]
    a2_dst = aout_ref[:, NCLASS:]
    ones1 = _ones_col(NHID)
    ones2 = _ones_col(NCLASS)

    for b in range(B):
        xb = x_ref[b]
        h_all = jnp.dot(xb, wcat, preferred_element_type=jnp.float32)
        fg = jnp.dot(h_all, acomb,
                     preferred_element_type=jnp.float32)
        fgT = fg.T
        P = jnp.exp(-fg[:, :NHEADS]).astype(jnp.bfloat16)
        PA = jnp.exp(-ALPHA * fg[:, :NHEADS]).astype(jnp.bfloat16)
        Q = jnp.exp(-fgT[NHEADS:, :]).astype(jnp.bfloat16)
        QA = jnp.exp(-ALPHA * fgT[NHEADS:, :]).astype(jnp.bfloat16)

        head_outs = []
        for hi in range(NHEADS):
            h = h_all[:, hi * NHID:(hi + 1) * NHID]
            head_outs.append(_elu(_att_prop(
                P[:, hi:hi + 1], PA[:, hi:hi + 1],
                Q[hi:hi + 1, :], QA[hi:hi + 1, :], h, mask, ones1, NHID)))
        x1 = jnp.concatenate(head_outs, axis=1)

        h2 = jnp.dot(x1, wout_ref[...], preferred_element_type=jnp.float32)
        f2 = jax.lax.dot_general(h2, a2_src, _NT,
                                 preferred_element_type=jnp.float32)
        g2T = jax.lax.dot_general(a2_dst, h2, _NT,
                                  preferred_element_type=jnp.float32)
        out_ref[b] = _elu(_att_prop(
            jnp.exp(-f2).astype(jnp.bfloat16),
            jnp.exp(-ALPHA * f2).astype(jnp.bfloat16),
            jnp.exp(-g2T).astype(jnp.bfloat16),
            jnp.exp(-ALPHA * g2T).astype(jnp.bfloat16), h2, mask, ones2, NCLASS))


def kernel(x, adj, W_att, a_att, W_out, a_out):
    return pl.pallas_call(
        _gat_body,
        out_shape=jax.ShapeDtypeStruct((B, N, NCLASS), jnp.float32),
    )(x, adj, W_att, a_att, W_out, a_out)
